# baseline (speedup 1.0000x reference)
import functools

import numpy as np

import jax
import jax.numpy as jnp
from jax.experimental import pallas as pl
from jax.experimental.pallas import tpu as pltpu

_F32_TINY = float(np.finfo(np.float32).tiny)


def _threefry_w(base, tw, kk):
    row = jax.lax.broadcasted_iota(jnp.uint32, (tw, kk), 0)
    col = jax.lax.broadcasted_iota(jnp.uint32, (tw, kk), 1)
    i = base + row * jnp.uint32(kk) + col

    ks0 = jnp.uint32(0)
    ks1 = jnp.uint32(1)
    ks2 = jnp.uint32(0x1BD11BDB)

    x0 = jnp.zeros((tw, kk), jnp.uint32)
    x1 = i + ks1

    rot_a = (13, 15, 26, 6)
    rot_b = (17, 29, 16, 24)

    def rounds(x0, x1, rots):
        for r in rots:
            x0 = x0 + x1
            x1 = (x1 << r) | (x1 >> (32 - r))
            x1 = x1 ^ x0
        return x0, x1

    x0, x1 = rounds(x0, x1, rot_a)
    x0 = x0 + ks1
    x1 = x1 + ks2 + jnp.uint32(1)
    x0, x1 = rounds(x0, x1, rot_b)
    x0 = x0 + ks2
    x1 = x1 + ks0 + jnp.uint32(2)
    x0, x1 = rounds(x0, x1, rot_a)
    x0 = x0 + ks0
    x1 = x1 + ks1 + jnp.uint32(3)
    x0, x1 = rounds(x0, x1, rot_b)
    x0 = x0 + ks1
    x1 = x1 + ks2 + jnp.uint32(4)
    x0, x1 = rounds(x0, x1, rot_a)
    x0 = x0 + ks2
    x1 = x1 + ks0 + jnp.uint32(5)

    bits = x0 ^ x1
    fbits = (bits >> 9) | jnp.uint32(0x3F800000)
    f = jax.lax.bitcast_convert_type(fbits, jnp.float32) - 1.0
    u = jnp.maximum(jnp.float32(_F32_TINY), f)
    t = jnp.log(u)
    return 1.0 / (t * t)


def _vq_tile(z_ref, c_ref, zq_ref, hard_ref, kl_ref, cm_ref, *, log_k):
    z = z_ref[0]
    c = c_ref[...]
    tw = z.shape[1]
    kk = c.shape[0]

    b = pl.program_id(0)
    t = pl.program_id(1)
    nt = pl.num_programs(1)
    base = ((b * nt + t) * (tw * kk)).astype(jnp.uint32)
    wmat = _threefry_w(base, tw, kk)

    csq = jnp.sum(c * c, axis=1)
    zsq = jnp.sum(z * z, axis=0)
    zc = jax.lax.dot_general(z, c, (((0,), (1,)), ((), ())),
                             preferred_element_type=jnp.float32)
    logits = 2.0 * zc - zsq[:, None] - csq[None, :]

    m = jnp.max(logits, axis=1, keepdims=True)
    u = jnp.exp(logits - m)
    s_u = jnp.sum(u, axis=1)
    p_l = jnp.sum(u * logits, axis=1)

    v = (u * u) * wmat
    s_v = jnp.sum(v, axis=1)

    col = jax.lax.broadcasted_iota(jnp.int32, (tw, kk), 1)
    vmax = jnp.max(v, axis=1, keepdims=True)
    hard = jnp.min(jnp.where(v == vmax, col, kk), axis=1)
    hard_ref[0, 0, :] = hard.astype(jnp.int32)

    zq = jax.lax.dot_general(c, v, (((0,), (1,)), ((), ())),
                             preferred_element_type=jnp.float32)
    zq_ref[0] = zq * (1.0 / s_v)[None, :]

    exp_l = p_l / s_u
    kl_ref[0, 0, 0, 0] = jnp.sum(exp_l - m[:, 0] - jnp.log(s_u) + log_k)
    cm_ref[0, 0, 0, 0] = -jnp.sum(exp_l)


def kernel(z_e, codebook):
    bs, feat, w = z_e.shape
    k = codebook.shape[0]
    tw = min(w, 1024)
    nt = w // tw
    log_k = float(np.log(k))

    grid = (bs, nt)
    z_q, hard3, klp, cmp_ = pl.pallas_call(
        functools.partial(_vq_tile, log_k=log_k),
        compiler_params=pltpu.CompilerParams(
            dimension_semantics=("parallel", "parallel")),
        grid=grid,
        in_specs=[
            pl.BlockSpec((1, feat, tw), lambda b, t: (b, 0, t)),
            pl.BlockSpec((k, feat), lambda b, t: (0, 0)),
        ],
        out_specs=[
            pl.BlockSpec((1, feat, tw), lambda b, t: (b, 0, t)),
            pl.BlockSpec((1, 1, tw), lambda b, t: (b, 0, t)),
            pl.BlockSpec((1, 1, 1, 1), lambda b, t: (b, t, 0, 0),
                         memory_space=pltpu.SMEM),
            pl.BlockSpec((1, 1, 1, 1), lambda b, t: (b, t, 0, 0),
                         memory_space=pltpu.SMEM),
        ],
        out_shape=[
            jax.ShapeDtypeStruct((bs, feat, w), jnp.float32),
            jax.ShapeDtypeStruct((bs, 1, w), jnp.int32),
            jax.ShapeDtypeStruct((bs, nt, 1, 1), jnp.float32),
            jax.ShapeDtypeStruct((bs, nt, 1, 1), jnp.float32),
        ],
    )(z_e, codebook)

    hard_indices = hard3.reshape(bs, w)
    kl = jnp.sum(klp) / bs
    commit = jnp.sum(cmp_) / bs
    return (z_q, hard_indices, kl, commit)
